# baseline (speedup 1.0000x reference)
import functools

import jax
import jax.numpy as jnp
from jax import lax
from jax.experimental import pallas as pl
from jax.experimental.pallas import tpu as pltpu
from jax.experimental.pallas import tpu_sc as plsc

_VOCAB = 1000000
_DIM = 32
_B = 4096
_L = 20
_XDIM = 64
_OUT = _XDIM + _L * _DIM
_PACK = 4
_PROWS = _VOCAB // _PACK
_PW = _PACK * _DIM

_NC = 2
_NS = 16
_NW = _NC * _NS
_BPW = _B // _NW

_TFULL = _VOCAB // 128
_TAIL = _VOCAB - _TFULL * 128
_RPT = _TFULL // _NW

_COMPILER_PARAMS = pltpu.CompilerParams(
    use_tc_tiling_on_sc=True, needs_layout_passes=False)


def _mesh():
    return plsc.VectorSubcoreMesh(
        core_axis_name="c", subcore_axis_name="s",
        num_cores=_NC, num_subcores=_NS)


def _transpose_chunk(chunk_v, outc_v, i):
    base = 4 * i
    for p in range(8):
        cvec = jnp.full((16,), base + p // 2, jnp.int32)
        rvec = (p % 2) * 16 + lax.iota(jnp.int32, 16)
        v = plsc.load_gather(chunk_v, [rvec, cvec])
        outc_v[i, pl.ds(p * 16, 16)] = v


def _repack_body(table_t, tail_pk, tpk, chunk_v, outc_v):
    wid = lax.axis_index("s") * _NC + lax.axis_index("c")
    n_t = jnp.where(wid < 4, _RPT + 1, _RPT)
    t0 = wid * _RPT + jnp.minimum(wid, 4)

    def per_tile(ti, _):
        t = t0 + ti
        pltpu.sync_copy(table_t.at[:, pl.ds(t * 128, 128)], chunk_v)

        def per_row(i, _):
            _transpose_chunk(chunk_v, outc_v, i)
            return ()

        lax.fori_loop(0, 32, per_row, (), unroll=False)
        pltpu.sync_copy(outc_v, tpk.at[pl.ds(t * 32, 32)])
        return ()

    lax.fori_loop(0, n_t, per_tile, (), unroll=False)

    @pl.when(wid == _NW - 1)
    def _():
        pltpu.sync_copy(tail_pk, outc_v.at[pl.ds(0, _TAIL // 4)])
        pltpu.sync_copy(
            outc_v.at[pl.ds(0, _TAIL // 4)],
            tpk.at[pl.ds(_TFULL * 32, _TAIL // 4)])


@jax.jit
def _repack(table_t, tail_pk):
    f = functools.partial(
        pl.kernel,
        out_type=jax.ShapeDtypeStruct((_PROWS, _PW), jnp.float32),
        mesh=_mesh(),
        scratch_types=[
            pltpu.VMEM((_DIM, 128), jnp.float32),
            pltpu.VMEM((32, _PW), jnp.float32),
        ],
        compiler_params=_COMPILER_PARAMS,
    )(_repack_body)
    return f(table_t, tail_pk)


def _fused_body(tpk, embed_t, x_t, out_t, idx4_v, off_v, rows_v, slab, sem):
    wid = lax.axis_index("s") * _NC + lax.axis_index("c")
    col0 = wid * _BPW

    pltpu.sync_copy(x_t.at[:, pl.ds(col0, _BPW)], slab.at[pl.ds(0, _XDIM)])
    pltpu.sync_copy(embed_t.at[:, pl.ds(col0, _BPW)], idx4_v)

    def prep(t, _):
        l = t // (_BPW // 16)
        j = t % (_BPW // 16)
        v = idx4_v[l, pl.ds(j * 16, 16)]
        off_v[l, pl.ds(j * 16, 16)] = (v & 3) * _DIM
        idx4_v[l, pl.ds(j * 16, 16)] = v >> 2
        return ()

    lax.fori_loop(0, _L * (_BPW // 16), prep, (), unroll=False)

    def step(l, _):
        pltpu.async_copy(tpk.at[idx4_v.at[l]], rows_v, sem).wait()
        rbase = _XDIM + l * _DIM

        def inner(jj, _):
            off16 = off_v[l, pl.ds(jj * 16, 16)]
            jvec = jj * 16 + lax.iota(jnp.int32, 16)
            for rr in range(_DIM):
                v = plsc.load_gather(rows_v, [jvec, off16 + rr])
                slab[rbase + rr, pl.ds(jj * 16, 16)] = v
            return ()

        lax.fori_loop(0, _BPW // 16, inner, (), unroll=False)
        return ()

    lax.fori_loop(0, _L, step, (), unroll=False)
    pltpu.sync_copy(slab, out_t.at[:, pl.ds(col0, _BPW)])


@jax.jit
def _fused(tpk, embed_t, x_t):
    f = functools.partial(
        pl.kernel,
        out_type=jax.ShapeDtypeStruct((_OUT, _B), jnp.float32),
        mesh=_mesh(),
        scratch_types=[
            pltpu.VMEM((_L, _BPW), jnp.int32),
            pltpu.VMEM((_L, _BPW), jnp.int32),
            pltpu.VMEM((_BPW, _PW), jnp.float32),
            pltpu.VMEM((_OUT, _BPW), jnp.float32),
            pltpu.SemaphoreType.DMA,
        ],
        compiler_params=_COMPILER_PARAMS,
    )(_fused_body)
    return f(tpk, embed_t, x_t)


def kernel(X, embed, table0):
    table_t = jnp.swapaxes(table0, 0, 1)
    tail_pk = jnp.reshape(
        lax.slice(table0, (_TFULL * 128, 0), (_VOCAB, _DIM)), (_TAIL // 4, _PW))
    tpk = _repack(table_t, tail_pk)
    embed_t = jnp.swapaxes(embed[0], 0, 1)
    x_t = jnp.swapaxes(X, 0, 1)
    out_t = _fused(tpk, embed_t, x_t)
    return jnp.swapaxes(out_t, 0, 1)


# speedup vs baseline: 1.2043x; 1.2043x over previous
import functools

import jax
import jax.numpy as jnp
from jax import lax
from jax.experimental import pallas as pl
from jax.experimental.pallas import tpu as pltpu
from jax.experimental.pallas import tpu_sc as plsc

_VOCAB = 1000000
_DIM = 32
_B = 4096
_L = 20
_XDIM = 64
_OUT = _XDIM + _L * _DIM
_PACK = 4
_PROWS = _VOCAB // _PACK
_PW = _PACK * _DIM

_NC = 2
_NS = 16
_NW = _NC * _NS
_BPW = _B // _NW

_HSLAB = _OUT // 2
_TFULL = _VOCAB // 128
_TAIL = _VOCAB - _TFULL * 128
_RPT = _TFULL // _NW

_COMPILER_PARAMS = pltpu.CompilerParams(
    use_tc_tiling_on_sc=True, needs_layout_passes=False)


def _mesh():
    return plsc.VectorSubcoreMesh(
        core_axis_name="c", subcore_axis_name="s",
        num_cores=_NC, num_subcores=_NS)


def _transpose_tile(chunk, outc, rvecs):
    for i in range(32):
        for p in range(8):
            cvec = jnp.full((16,), 4 * i + p // 2, jnp.int32)
            v = plsc.load_gather(chunk, [rvecs[p % 2], cvec])
            outc[i, pl.ds(p * 16, 16)] = v


def _repack_body(table_t, tail_pk, tpk, chunk_v, outc_v, isem, osem):
    wid = lax.axis_index("s") * _NC + lax.axis_index("c")
    n_t = jnp.where(wid < 4, _RPT + 1, _RPT)
    t0 = wid * _RPT + jnp.minimum(wid, 4)
    rvecs = (lax.iota(jnp.int32, 16), 16 + lax.iota(jnp.int32, 16))

    def start_in(t, par):
        pltpu.async_copy(
            table_t.at[:, pl.ds(t * 128, 128)], chunk_v.at[par], isem)

    start_in(t0, 0)
    start_in(t0 + 1, 1)

    def step(ti, par):
        t = t0 + ti
        pltpu.make_async_copy(
            table_t.at[:, pl.ds(t * 128, 128)], chunk_v.at[par], isem).wait()
        @pl.when(ti >= 2)
        def _():
            pltpu.make_async_copy(
                outc_v.at[par], tpk.at[pl.ds(t * 32, 32)], osem).wait()
        _transpose_tile(chunk_v.at[par], outc_v.at[par], rvecs)

        @pl.when(ti + 2 < n_t)
        def _():
            start_in(t + 2, par)
        pltpu.async_copy(outc_v.at[par], tpk.at[pl.ds(t * 32, 32)], osem)
        return ()

    def pair(k, _):
        step(2 * k, 0)
        step(2 * k + 1, 1)
        return ()

    lax.fori_loop(0, n_t // 2, pair, (), unroll=False)

    @pl.when(n_t % 2 == 1)
    def _():
        step(n_t - 1, 0)

    pltpu.make_async_copy(outc_v.at[0], tpk.at[pl.ds(0, 32)], osem).wait()
    pltpu.make_async_copy(outc_v.at[0], tpk.at[pl.ds(0, 32)], osem).wait()

    @pl.when(wid == _NW - 1)
    def _():
        pltpu.sync_copy(tail_pk, outc_v.at[0, pl.ds(0, _TAIL // 4)])
        pltpu.sync_copy(
            outc_v.at[0, pl.ds(0, _TAIL // 4)],
            tpk.at[pl.ds(_TFULL * 32, _TAIL // 4)])


@jax.jit
def _repack(table_t, tail_pk):
    f = functools.partial(
        pl.kernel,
        out_type=jax.ShapeDtypeStruct((_PROWS, _PW), jnp.float32),
        mesh=_mesh(),
        scratch_types=[
            pltpu.VMEM((2, _DIM, 128), jnp.float32),
            pltpu.VMEM((2, 32, _PW), jnp.float32),
            pltpu.SemaphoreType.DMA,
            pltpu.SemaphoreType.DMA,
        ],
        compiler_params=_COMPILER_PARAMS,
    )(_repack_body)
    return f(table_t, tail_pk)


def _fused_body(tpk, embed_t, x_t, out_t, idx4_v, off_v, rows_v, slab, sem):
    wid = lax.axis_index("s") * _NC + lax.axis_index("c")
    col0 = wid * _BPW

    pltpu.sync_copy(x_t.at[:, pl.ds(col0, _BPW)], slab.at[pl.ds(0, _XDIM)])
    pltpu.sync_copy(embed_t.at[:, pl.ds(col0, _BPW)], idx4_v)

    def prep(t, _):
        l = t // (_BPW // 16)
        j = t % (_BPW // 16)
        v = idx4_v[l, pl.ds(j * 16, 16)]
        off_v[l, pl.ds(j * 16, 16)] = (v & 3) * _DIM
        idx4_v[l, pl.ds(j * 16, 16)] = v >> 2
        return ()

    lax.fori_loop(0, _L * (_BPW // 16), prep, (), unroll=False)

    def fire(l, par):
        pltpu.async_copy(tpk.at[idx4_v.at[l]], rows_v.at[par], sem)

    fire(0, 0)
    fire(1, 1)

    jvecs = [jj * 16 + lax.iota(jnp.int32, 16) for jj in range(_BPW // 16)]

    def step(l, par, row_off):
        pltpu.make_async_copy(
            tpk.at[idx4_v.at[l]], rows_v.at[par], sem).wait()
        rbase = _XDIM + l * _DIM - row_off
        for jj in range(_BPW // 16):
            off16 = off_v[l, pl.ds(jj * 16, 16)]
            for rr in range(_DIM):
                v = plsc.load_gather(rows_v.at[par], [jvecs[jj], off16 + rr])
                slab[rbase + rr, pl.ds(jj * 16, 16)] = v

        @pl.when(l + 2 < _L)
        def _():
            fire(l + 2, par)
        return ()

    def pair_a(k, _):
        step(2 * k, 0, 0)
        step(2 * k + 1, 1, 0)
        return ()

    lax.fori_loop(0, 4, pair_a, (), unroll=False)
    step(8, 0, 0)
    pltpu.sync_copy(slab, out_t.at[pl.ds(0, _HSLAB), pl.ds(col0, _BPW)])

    step(9, 1, _HSLAB)

    def pair_b(k, _):
        step(10 + 2 * k, 0, _HSLAB)
        step(11 + 2 * k, 1, _HSLAB)
        return ()

    lax.fori_loop(0, 5, pair_b, (), unroll=False)
    pltpu.sync_copy(slab, out_t.at[pl.ds(_HSLAB, _HSLAB), pl.ds(col0, _BPW)])


@jax.jit
def _fused(tpk, embed_t, x_t):
    f = functools.partial(
        pl.kernel,
        out_type=jax.ShapeDtypeStruct((_OUT, _B), jnp.float32),
        mesh=_mesh(),
        scratch_types=[
            pltpu.VMEM((_L, _BPW), jnp.int32),
            pltpu.VMEM((_L, _BPW), jnp.int32),
            pltpu.VMEM((2, _BPW, _PW), jnp.float32),
            pltpu.VMEM((_HSLAB, _BPW), jnp.float32),
            pltpu.SemaphoreType.DMA,
        ],
        compiler_params=_COMPILER_PARAMS,
    )(_fused_body)
    return f(tpk, embed_t, x_t)


def kernel(X, embed, table0):
    table_t = jnp.swapaxes(table0, 0, 1)
    tail_pk = jnp.reshape(
        lax.slice(table0, (_TFULL * 128, 0), (_VOCAB, _DIM)), (_TAIL // 4, _PW))
    tpk = _repack(table_t, tail_pk)
    embed_t = jnp.swapaxes(embed[0], 0, 1)
    x_t = jnp.swapaxes(X, 0, 1)
    out_t = _fused(tpk, embed_t, x_t)
    return jnp.swapaxes(out_t, 0, 1)


# speedup vs baseline: 2.4155x; 2.0057x over previous
import functools

import jax
import jax.numpy as jnp
from jax import lax
from jax.experimental import pallas as pl
from jax.experimental.pallas import tpu as pltpu
from jax.experimental.pallas import tpu_sc as plsc

_VOCAB = 1000000
_DIM = 32
_B = 4096
_L = 20
_XDIM = 64
_OUT = _XDIM + _L * _DIM
_PACK = 4
_PROWS = _VOCAB // _PACK
_PW = _PACK * _DIM

_NC = 2
_NS = 16
_NW = _NC * _NS
_BPW = _B // _NW

_HSLAB = _OUT // 2
_TFULL = _VOCAB // 128
_TAIL = _VOCAB - _TFULL * 128
_RPT = _TFULL // _NW

_COMPILER_PARAMS = pltpu.CompilerParams(
    use_tc_tiling_on_sc=True, needs_layout_passes=False)


def _mesh():
    return plsc.VectorSubcoreMesh(
        core_axis_name="c", subcore_axis_name="s",
        num_cores=_NC, num_subcores=_NS)


def _transpose_tile(chunk, outc, rvecs):
    @plsc.parallel_loop(0, 32, unroll=8)
    def _(i):
        cv = jnp.full((16,), 4 * i, jnp.int32)
        for h in range(4):
            cvec = cv + h
            for q in range(2):
                v = plsc.load_gather(chunk, [rvecs[q], cvec])
                outc[i, pl.ds((2 * h + q) * 16, 16)] = v


def _repack_body(table_t, tail_pk, tpk, chunk_v, outc_v, isem, osem):
    wid = lax.axis_index("s") * _NC + lax.axis_index("c")
    n_t = jnp.where(wid < 4, _RPT + 1, _RPT)
    t0 = wid * _RPT + jnp.minimum(wid, 4)
    rvecs = (lax.iota(jnp.int32, 16), 16 + lax.iota(jnp.int32, 16))

    def start_in(t, par):
        pltpu.async_copy(
            table_t.at[:, pl.ds(t * 128, 128)], chunk_v.at[par], isem)

    start_in(t0, 0)
    start_in(t0 + 1, 1)

    def step(ti, par):
        t = t0 + ti
        pltpu.make_async_copy(
            table_t.at[:, pl.ds(t * 128, 128)], chunk_v.at[par], isem).wait()
        @pl.when(ti >= 2)
        def _():
            pltpu.make_async_copy(
                outc_v.at[par], tpk.at[pl.ds(t * 32, 32)], osem).wait()
        _transpose_tile(chunk_v.at[par], outc_v.at[par], rvecs)

        @pl.when(ti + 2 < n_t)
        def _():
            start_in(t + 2, par)
        pltpu.async_copy(outc_v.at[par], tpk.at[pl.ds(t * 32, 32)], osem)
        return ()

    def pair(k, _):
        step(2 * k, 0)
        step(2 * k + 1, 1)
        return ()

    lax.fori_loop(0, n_t // 2, pair, (), unroll=False)

    @pl.when(n_t % 2 == 1)
    def _():
        step(n_t - 1, 0)

    pltpu.make_async_copy(outc_v.at[0], tpk.at[pl.ds(0, 32)], osem).wait()
    pltpu.make_async_copy(outc_v.at[0], tpk.at[pl.ds(0, 32)], osem).wait()

    @pl.when(wid == _NW - 1)
    def _():
        pltpu.sync_copy(tail_pk, outc_v.at[0, pl.ds(0, _TAIL // 4)])
        pltpu.sync_copy(
            outc_v.at[0, pl.ds(0, _TAIL // 4)],
            tpk.at[pl.ds(_TFULL * 32, _TAIL // 4)])


@jax.jit
def _repack(table_t, tail_pk):
    f = functools.partial(
        pl.kernel,
        out_type=jax.ShapeDtypeStruct((_PROWS, _PW), jnp.float32),
        mesh=_mesh(),
        scratch_types=[
            pltpu.VMEM((2, _DIM, 128), jnp.float32),
            pltpu.VMEM((2, 32, _PW), jnp.float32),
            pltpu.SemaphoreType.DMA,
            pltpu.SemaphoreType.DMA,
        ],
        compiler_params=_COMPILER_PARAMS,
    )(_repack_body)
    return f(table_t, tail_pk)


def _fused_body(tpk, embed_t, x_t, out_t, idx4_v, off_v, rows_v, slab, sem):
    wid = lax.axis_index("s") * _NC + lax.axis_index("c")
    col0 = wid * _BPW

    pltpu.sync_copy(x_t.at[:, pl.ds(col0, _BPW)], slab.at[pl.ds(0, _XDIM)])
    pltpu.sync_copy(embed_t.at[:, pl.ds(col0, _BPW)], idx4_v)

    def prep(t, _):
        l = t // (_BPW // 16)
        j = t % (_BPW // 16)
        v = idx4_v[l, pl.ds(j * 16, 16)]
        off_v[l, pl.ds(j * 16, 16)] = (v & 3) * _DIM
        idx4_v[l, pl.ds(j * 16, 16)] = v >> 2
        return ()

    lax.fori_loop(0, _L * (_BPW // 16), prep, (), unroll=False)

    def fire(l, par):
        pltpu.async_copy(tpk.at[idx4_v.at[l]], rows_v.at[par], sem)

    fire(0, 0)
    fire(1, 1)

    jvecs = [jj * 16 + lax.iota(jnp.int32, 16) for jj in range(_BPW // 16)]

    def step(l, par, row_off):
        pltpu.make_async_copy(
            tpk.at[idx4_v.at[l]], rows_v.at[par], sem).wait()
        rbase = _XDIM + l * _DIM - row_off
        for jj in range(_BPW // 16):
            off16 = off_v[l, pl.ds(jj * 16, 16)]

            @plsc.parallel_loop(0, _DIM, unroll=8)
            def _(rr):
                v = plsc.load_gather(rows_v.at[par], [jvecs[jj], off16 + rr])
                slab[rbase + rr, pl.ds(jj * 16, 16)] = v

        @pl.when(l + 2 < _L)
        def _():
            fire(l + 2, par)
        return ()

    def pair_a(k, _):
        step(2 * k, 0, 0)
        step(2 * k + 1, 1, 0)
        return ()

    lax.fori_loop(0, 4, pair_a, (), unroll=False)
    step(8, 0, 0)
    pltpu.sync_copy(slab, out_t.at[pl.ds(0, _HSLAB), pl.ds(col0, _BPW)])

    step(9, 1, _HSLAB)

    def pair_b(k, _):
        step(10 + 2 * k, 0, _HSLAB)
        step(11 + 2 * k, 1, _HSLAB)
        return ()

    lax.fori_loop(0, 5, pair_b, (), unroll=False)
    pltpu.sync_copy(slab, out_t.at[pl.ds(_HSLAB, _HSLAB), pl.ds(col0, _BPW)])


@jax.jit
def _fused(tpk, embed_t, x_t):
    f = functools.partial(
        pl.kernel,
        out_type=jax.ShapeDtypeStruct((_OUT, _B), jnp.float32),
        mesh=_mesh(),
        scratch_types=[
            pltpu.VMEM((_L, _BPW), jnp.int32),
            pltpu.VMEM((_L, _BPW), jnp.int32),
            pltpu.VMEM((2, _BPW, _PW), jnp.float32),
            pltpu.VMEM((_HSLAB, _BPW), jnp.float32),
            pltpu.SemaphoreType.DMA,
        ],
        compiler_params=_COMPILER_PARAMS,
    )(_fused_body)
    return f(tpk, embed_t, x_t)


def kernel(X, embed, table0):
    table_t = jnp.swapaxes(table0, 0, 1)
    tail_pk = jnp.reshape(
        lax.slice(table0, (_TFULL * 128, 0), (_VOCAB, _DIM)), (_TAIL // 4, _PW))
    tpk = _repack(table_t, tail_pk)
    embed_t = jnp.swapaxes(embed[0], 0, 1)
    x_t = jnp.swapaxes(X, 0, 1)
    out_t = _fused(tpk, embed_t, x_t)
    return jnp.swapaxes(out_t, 0, 1)


# speedup vs baseline: 4.6689x; 1.9329x over previous
import functools

import jax
import jax.numpy as jnp
from jax import lax
from jax.experimental import pallas as pl
from jax.experimental.pallas import tpu as pltpu
from jax.experimental.pallas import tpu_sc as plsc

_VOCAB = 1000000
_DIM = 32
_B = 4096
_L = 20
_XDIM = 64
_OUT = _XDIM + _L * _DIM
_PACK = 4
_PROWS = _VOCAB // _PACK
_PW = _PACK * _DIM

_NC = 2
_NS = 16
_NW = _NC * _NS
_BPW = _B // _NW

_HSLAB = _OUT // 2
_TFULL = _VOCAB // 128
_TAIL = _VOCAB - _TFULL * 128
_RPT = _TFULL // _NW

_COMPILER_PARAMS = pltpu.CompilerParams(
    use_tc_tiling_on_sc=True, needs_layout_passes=False)


def _mesh():
    return plsc.VectorSubcoreMesh(
        core_axis_name="c", subcore_axis_name="s",
        num_cores=_NC, num_subcores=_NS)


def _transpose_tile(chunk, skew, outc, rvecs):
    iota = rvecs[0]

    @plsc.parallel_loop(0, 32, unroll=8)
    def _(c):
        rot = (iota + c) & 15
        cvec = jnp.full((16,), c, jnp.int32)
        for b in range(8):
            v = chunk[c, pl.ds(b * 16, 16)]
            plsc.store_scatter(skew, [cvec, b * 16 + rot], v)

    @plsc.parallel_loop(0, 32, unroll=8)
    def _(i):
        for h in range(4):
            j = 4 * i + h
            jhi = j & ~15
            for q in range(2):
                rvec = rvecs[q]
                lane = ((j + rvec) & 15) + jhi
                v = plsc.load_gather(skew, [rvec, lane])
                outc[i, pl.ds((2 * h + q) * 16, 16)] = v


def _repack_body(table_t, tail_pk, tpk, chunk_v, skew_v, outc_v, isem, osem):
    wid = lax.axis_index("s") * _NC + lax.axis_index("c")
    n_t = jnp.where(wid < 4, _RPT + 1, _RPT)
    t0 = wid * _RPT + jnp.minimum(wid, 4)
    rvecs = (lax.iota(jnp.int32, 16), 16 + lax.iota(jnp.int32, 16))

    def start_in(t, par):
        pltpu.async_copy(
            table_t.at[:, pl.ds(t * 128, 128)], chunk_v.at[par], isem)

    start_in(t0, 0)
    start_in(t0 + 1, 1)

    def step(ti, par):
        t = t0 + ti
        pltpu.make_async_copy(
            table_t.at[:, pl.ds(t * 128, 128)], chunk_v.at[par], isem).wait()
        @pl.when(ti >= 2)
        def _():
            pltpu.make_async_copy(
                outc_v.at[par], tpk.at[pl.ds(t * 32, 32)], osem).wait()
        _transpose_tile(chunk_v.at[par], skew_v, outc_v.at[par], rvecs)

        @pl.when(ti + 2 < n_t)
        def _():
            start_in(t + 2, par)
        pltpu.async_copy(outc_v.at[par], tpk.at[pl.ds(t * 32, 32)], osem)
        return ()

    def pair(k, _):
        step(2 * k, 0)
        step(2 * k + 1, 1)
        return ()

    lax.fori_loop(0, n_t // 2, pair, (), unroll=False)

    @pl.when(n_t % 2 == 1)
    def _():
        step(n_t - 1, 0)

    pltpu.make_async_copy(outc_v.at[0], tpk.at[pl.ds(0, 32)], osem).wait()
    pltpu.make_async_copy(outc_v.at[0], tpk.at[pl.ds(0, 32)], osem).wait()

    @pl.when(wid == _NW - 1)
    def _():
        pltpu.sync_copy(tail_pk, outc_v.at[0, pl.ds(0, _TAIL // 4)])
        pltpu.sync_copy(
            outc_v.at[0, pl.ds(0, _TAIL // 4)],
            tpk.at[pl.ds(_TFULL * 32, _TAIL // 4)])


@jax.jit
def _repack(table_t, tail_pk):
    f = functools.partial(
        pl.kernel,
        out_type=jax.ShapeDtypeStruct((_PROWS, _PW), jnp.float32),
        mesh=_mesh(),
        scratch_types=[
            pltpu.VMEM((2, _DIM, 128), jnp.float32),
            pltpu.VMEM((_DIM, 128), jnp.float32),
            pltpu.VMEM((2, 32, _PW), jnp.float32),
            pltpu.SemaphoreType.DMA,
            pltpu.SemaphoreType.DMA,
        ],
        compiler_params=_COMPILER_PARAMS,
    )(_repack_body)
    return f(table_t, tail_pk)


def _fused_body(tpk, embed_t, x_t, out_t, idx4_v, off_v, rows_v, slab, sem):
    wid = lax.axis_index("s") * _NC + lax.axis_index("c")
    col0 = wid * _BPW

    pltpu.sync_copy(x_t.at[:, pl.ds(col0, _BPW)], slab.at[pl.ds(0, _XDIM)])
    pltpu.sync_copy(embed_t.at[:, pl.ds(col0, _BPW)], idx4_v)

    def prep(t, _):
        l = t // (_BPW // 16)
        j = t % (_BPW // 16)
        v = idx4_v[l, pl.ds(j * 16, 16)]
        off_v[l, pl.ds(j * 16, 16)] = (v & 3) * _DIM
        idx4_v[l, pl.ds(j * 16, 16)] = v >> 2
        return ()

    lax.fori_loop(0, _L * (_BPW // 16), prep, (), unroll=False)

    def fire(l, par):
        pltpu.async_copy(tpk.at[idx4_v.at[l]], rows_v.at[par], sem)

    fire(0, 0)
    fire(1, 1)

    jvecs = [jj * 16 + lax.iota(jnp.int32, 16) for jj in range(_BPW // 16)]

    def step(l, par, row_off):
        pltpu.make_async_copy(
            tpk.at[idx4_v.at[l]], rows_v.at[par], sem).wait()
        rbase = _XDIM + l * _DIM - row_off
        for jj in range(_BPW // 16):
            off16 = off_v[l, pl.ds(jj * 16, 16)]

            @plsc.parallel_loop(0, _DIM, unroll=8)
            def _(rr):
                v = plsc.load_gather(rows_v.at[par], [jvecs[jj], off16 + rr])
                slab[rbase + rr, pl.ds(jj * 16, 16)] = v

        @pl.when(l + 2 < _L)
        def _():
            fire(l + 2, par)
        return ()

    def pair_a(k, _):
        step(2 * k, 0, 0)
        step(2 * k + 1, 1, 0)
        return ()

    lax.fori_loop(0, 4, pair_a, (), unroll=False)
    step(8, 0, 0)
    pltpu.sync_copy(slab, out_t.at[pl.ds(0, _HSLAB), pl.ds(col0, _BPW)])

    step(9, 1, _HSLAB)

    def pair_b(k, _):
        step(10 + 2 * k, 0, _HSLAB)
        step(11 + 2 * k, 1, _HSLAB)
        return ()

    lax.fori_loop(0, 5, pair_b, (), unroll=False)
    pltpu.sync_copy(slab, out_t.at[pl.ds(_HSLAB, _HSLAB), pl.ds(col0, _BPW)])


@jax.jit
def _fused(tpk, embed_t, x_t):
    f = functools.partial(
        pl.kernel,
        out_type=jax.ShapeDtypeStruct((_OUT, _B), jnp.float32),
        mesh=_mesh(),
        scratch_types=[
            pltpu.VMEM((_L, _BPW), jnp.int32),
            pltpu.VMEM((_L, _BPW), jnp.int32),
            pltpu.VMEM((2, _BPW, _PW), jnp.float32),
            pltpu.VMEM((_HSLAB, _BPW), jnp.float32),
            pltpu.SemaphoreType.DMA,
        ],
        compiler_params=_COMPILER_PARAMS,
    )(_fused_body)
    return f(tpk, embed_t, x_t)


def kernel(X, embed, table0):
    table_t = jnp.swapaxes(table0, 0, 1)
    tail_pk = jnp.reshape(
        lax.slice(table0, (_TFULL * 128, 0), (_VOCAB, _DIM)), (_TAIL // 4, _PW))
    tpk = _repack(table_t, tail_pk)
    embed_t = jnp.swapaxes(embed[0], 0, 1)
    x_t = jnp.swapaxes(X, 0, 1)
    out_t = _fused(tpk, embed_t, x_t)
    return jnp.swapaxes(out_t, 0, 1)


# speedup vs baseline: 5.8750x; 1.2583x over previous
import functools

import jax
import jax.numpy as jnp
from jax import lax
from jax.experimental import pallas as pl
from jax.experimental.pallas import tpu as pltpu
from jax.experimental.pallas import tpu_sc as plsc

_VOCAB = 1000000
_DIM = 32
_B = 4096
_L = 20
_XDIM = 64
_OUT = _XDIM + _L * _DIM
_PACK = 4
_PROWS = _VOCAB // _PACK
_PW = _PACK * _DIM

_NC = 2
_NS = 16
_NW = _NC * _NS
_BPW = _B // _NW

_HSLAB = _OUT // 2
_CHUNK = 256
_NFULL = _VOCAB // _CHUNK
_TAIL = _VOCAB - _NFULL * _CHUNK
_RPT = _NFULL // _NW

_COMPILER_PARAMS = pltpu.CompilerParams(
    use_tc_tiling_on_sc=True, needs_layout_passes=False)


def _mesh():
    return plsc.VectorSubcoreMesh(
        core_axis_name="c", subcore_axis_name="s",
        num_cores=_NC, num_subcores=_NS)


def _transpose_tile(chunk, skew, outc, rvecs):
    iota = rvecs[0]

    @plsc.parallel_loop(0, 32, unroll=8)
    def _(c):
        rot = (iota + c) & 15
        cvec = jnp.full((16,), c, jnp.int32)
        for b in range(_CHUNK // 16):
            v = chunk[c, pl.ds(b * 16, 16)]
            plsc.store_scatter(skew, [cvec, b * 16 + rot], v)

    @plsc.parallel_loop(0, _CHUNK // 4, unroll=8)
    def _(i):
        for h in range(4):
            j = 4 * i + h
            jhi = j & ~15
            for q in range(2):
                rvec = rvecs[q]
                lane = ((j + rvec) & 15) + jhi
                v = plsc.load_gather(skew, [rvec, lane])
                outc[i, pl.ds((2 * h + q) * 16, 16)] = v


def _repack_body(table_t, tail_pk, tpk, chunk_v, skew_v, outc_v, isem, osem):
    wid = lax.axis_index("s") * _NC + lax.axis_index("c")
    n_t = jnp.where(wid < 2, _RPT + 1, _RPT)
    t0 = wid * _RPT + jnp.minimum(wid, 2)
    rvecs = (lax.iota(jnp.int32, 16), 16 + lax.iota(jnp.int32, 16))

    def start_in(t, par):
        pltpu.async_copy(
            table_t.at[:, pl.ds(t * _CHUNK, _CHUNK)], chunk_v.at[par], isem)

    start_in(t0, 0)
    start_in(t0 + 1, 1)

    def step(ti, par):
        t = t0 + ti
        pltpu.make_async_copy(
            table_t.at[:, pl.ds(t * _CHUNK, _CHUNK)], chunk_v.at[par], isem).wait()
        @pl.when(ti >= 2)
        def _():
            pltpu.make_async_copy(
                outc_v.at[par], tpk.at[pl.ds(t * (_CHUNK // 4), _CHUNK // 4)], osem).wait()
        _transpose_tile(chunk_v.at[par], skew_v, outc_v.at[par], rvecs)

        @pl.when(ti + 2 < n_t)
        def _():
            start_in(t + 2, par)
        pltpu.async_copy(outc_v.at[par], tpk.at[pl.ds(t * (_CHUNK // 4), _CHUNK // 4)], osem)
        return ()

    def pair(k, _):
        step(2 * k, 0)
        step(2 * k + 1, 1)
        return ()

    lax.fori_loop(0, n_t // 2, pair, (), unroll=False)

    @pl.when(n_t % 2 == 1)
    def _():
        step(n_t - 1, 0)

    pltpu.make_async_copy(outc_v.at[0], tpk.at[pl.ds(0, _CHUNK // 4)], osem).wait()
    pltpu.make_async_copy(outc_v.at[0], tpk.at[pl.ds(0, _CHUNK // 4)], osem).wait()

    @pl.when(wid == _NW - 1)
    def _():
        pltpu.sync_copy(tail_pk, outc_v.at[0, pl.ds(0, _TAIL // 4)])
        pltpu.sync_copy(
            outc_v.at[0, pl.ds(0, _TAIL // 4)],
            tpk.at[pl.ds(_NFULL * (_CHUNK // 4), _TAIL // 4)])


@jax.jit
def _repack(table_t, tail_pk):
    f = functools.partial(
        pl.kernel,
        out_type=jax.ShapeDtypeStruct((_PROWS, _PW), jnp.float32),
        mesh=_mesh(),
        scratch_types=[
            pltpu.VMEM((2, _DIM, _CHUNK), jnp.float32),
            pltpu.VMEM((_DIM, _CHUNK), jnp.float32),
            pltpu.VMEM((2, _CHUNK // 4, _PW), jnp.float32),
            pltpu.SemaphoreType.DMA,
            pltpu.SemaphoreType.DMA,
        ],
        compiler_params=_COMPILER_PARAMS,
    )(_repack_body)
    return f(table_t, tail_pk)


def _fused_body(tpk, embed_t, x_t, out_t, idx4_v, off_v, rows_v, slab, sem):
    wid = lax.axis_index("s") * _NC + lax.axis_index("c")
    col0 = wid * _BPW

    pltpu.sync_copy(x_t.at[:, pl.ds(col0, _BPW)], slab.at[pl.ds(0, _XDIM)])
    pltpu.sync_copy(embed_t.at[:, pl.ds(col0, _BPW)], idx4_v)

    def prep(t, _):
        l = t // (_BPW // 16)
        j = t % (_BPW // 16)
        v = idx4_v[l, pl.ds(j * 16, 16)]
        off_v[l, pl.ds(j * 16, 16)] = (v & 3) * _DIM
        idx4_v[l, pl.ds(j * 16, 16)] = v >> 2
        return ()

    lax.fori_loop(0, _L * (_BPW // 16), prep, (), unroll=False)

    def fire(l, par):
        pltpu.async_copy(tpk.at[idx4_v.at[l]], rows_v.at[par], sem)

    fire(0, 0)
    fire(1, 1)

    jvecs = [jj * 16 + lax.iota(jnp.int32, 16) for jj in range(_BPW // 16)]

    def step(l, par, row_off):
        pltpu.make_async_copy(
            tpk.at[idx4_v.at[l]], rows_v.at[par], sem).wait()
        rbase = _XDIM + l * _DIM - row_off
        for jj in range(_BPW // 16):
            off16 = off_v[l, pl.ds(jj * 16, 16)]

            @plsc.parallel_loop(0, _DIM, unroll=8)
            def _(rr):
                v = plsc.load_gather(rows_v.at[par], [jvecs[jj], off16 + rr])
                slab[rbase + rr, pl.ds(jj * 16, 16)] = v

        @pl.when(l + 2 < _L)
        def _():
            fire(l + 2, par)
        return ()

    def pair_a(k, _):
        step(2 * k, 0, 0)
        step(2 * k + 1, 1, 0)
        return ()

    lax.fori_loop(0, 4, pair_a, (), unroll=False)
    step(8, 0, 0)
    pltpu.sync_copy(slab, out_t.at[pl.ds(0, _HSLAB), pl.ds(col0, _BPW)])

    step(9, 1, _HSLAB)

    def pair_b(k, _):
        step(10 + 2 * k, 0, _HSLAB)
        step(11 + 2 * k, 1, _HSLAB)
        return ()

    lax.fori_loop(0, 5, pair_b, (), unroll=False)
    pltpu.sync_copy(slab, out_t.at[pl.ds(_HSLAB, _HSLAB), pl.ds(col0, _BPW)])


@jax.jit
def _fused(tpk, embed_t, x_t):
    f = functools.partial(
        pl.kernel,
        out_type=jax.ShapeDtypeStruct((_OUT, _B), jnp.float32),
        mesh=_mesh(),
        scratch_types=[
            pltpu.VMEM((_L, _BPW), jnp.int32),
            pltpu.VMEM((_L, _BPW), jnp.int32),
            pltpu.VMEM((2, _BPW, _PW), jnp.float32),
            pltpu.VMEM((_HSLAB, _BPW), jnp.float32),
            pltpu.SemaphoreType.DMA,
        ],
        compiler_params=_COMPILER_PARAMS,
    )(_fused_body)
    return f(tpk, embed_t, x_t)


def kernel(X, embed, table0):
    table_t = jnp.swapaxes(table0, 0, 1)
    tail_pk = jnp.reshape(
        lax.slice(table0, (_NFULL * _CHUNK, 0), (_VOCAB, _DIM)), (_TAIL // 4, _PW))
    tpk = _repack(table_t, tail_pk)
    embed_t = jnp.swapaxes(embed[0], 0, 1)
    x_t = jnp.swapaxes(X, 0, 1)
    out_t = _fused(tpk, embed_t, x_t)
    return jnp.swapaxes(out_t, 0, 1)


# speedup vs baseline: 6.2680x; 1.0669x over previous
import functools

import jax
import jax.numpy as jnp
from jax import lax
from jax.experimental import pallas as pl
from jax.experimental.pallas import tpu as pltpu
from jax.experimental.pallas import tpu_sc as plsc

_VOCAB = 1000000
_DIM = 32
_B = 4096
_L = 20
_XDIM = 64
_OUT = _XDIM + _L * _DIM
_PACK = 4
_PROWS = _VOCAB // _PACK
_PW = _PACK * _DIM

_NC = 2
_NS = 16
_NW = _NC * _NS
_BPW = _B // _NW

_HSLAB = _OUT // 2
_CHUNK = 512
_NFULL = _VOCAB // _CHUNK
_TAIL = _VOCAB - _NFULL * _CHUNK
_RPT = _NFULL // _NW

_COMPILER_PARAMS = pltpu.CompilerParams(
    use_tc_tiling_on_sc=True, needs_layout_passes=False)


def _mesh():
    return plsc.VectorSubcoreMesh(
        core_axis_name="c", subcore_axis_name="s",
        num_cores=_NC, num_subcores=_NS)


def _transpose_tile(chunk, skew, outc, rvecs):
    iota = rvecs[0]

    @plsc.parallel_loop(0, 32, unroll=8)
    def _(c):
        rot = (iota + c) & 15
        cvec = jnp.full((16,), c, jnp.int32)
        for b in range(_CHUNK // 16):
            v = chunk[c, pl.ds(b * 16, 16)]
            plsc.store_scatter(skew, [cvec, b * 16 + rot], v)

    @plsc.parallel_loop(0, _CHUNK // 4, unroll=8)
    def _(i):
        for h in range(4):
            j = 4 * i + h
            jhi = j & ~15
            for q in range(2):
                rvec = rvecs[q]
                lane = ((j + rvec) & 15) + jhi
                v = plsc.load_gather(skew, [rvec, lane])
                outc[i, pl.ds((2 * h + q) * 16, 16)] = v


def _repack_body(table_t, tail_pk, tpk, chunk_v, skew_v, outc_v, isem, osem):
    wid = lax.axis_index("s") * _NC + lax.axis_index("c")
    n_t = jnp.where(wid < 1, _RPT + 1, _RPT)
    t0 = wid * _RPT + jnp.minimum(wid, 1)
    rvecs = (lax.iota(jnp.int32, 16), 16 + lax.iota(jnp.int32, 16))

    def start_in(t, par):
        pltpu.async_copy(
            table_t.at[:, pl.ds(t * _CHUNK, _CHUNK)], chunk_v.at[par], isem)

    start_in(t0, 0)
    start_in(t0 + 1, 1)

    def step(ti, par):
        t = t0 + ti
        pltpu.make_async_copy(
            table_t.at[:, pl.ds(t * _CHUNK, _CHUNK)], chunk_v.at[par], isem).wait()
        @pl.when(ti >= 2)
        def _():
            pltpu.make_async_copy(
                outc_v.at[par], tpk.at[pl.ds(t * (_CHUNK // 4), _CHUNK // 4)], osem).wait()
        _transpose_tile(chunk_v.at[par], skew_v, outc_v.at[par], rvecs)

        @pl.when(ti + 2 < n_t)
        def _():
            start_in(t + 2, par)
        pltpu.async_copy(outc_v.at[par], tpk.at[pl.ds(t * (_CHUNK // 4), _CHUNK // 4)], osem)
        return ()

    def pair(k, _):
        step(2 * k, 0)
        step(2 * k + 1, 1)
        return ()

    lax.fori_loop(0, n_t // 2, pair, (), unroll=False)

    @pl.when(n_t % 2 == 1)
    def _():
        step(n_t - 1, 0)

    pltpu.make_async_copy(outc_v.at[0], tpk.at[pl.ds(0, _CHUNK // 4)], osem).wait()
    pltpu.make_async_copy(outc_v.at[0], tpk.at[pl.ds(0, _CHUNK // 4)], osem).wait()

    @pl.when(wid == _NW - 1)
    def _():
        pltpu.sync_copy(tail_pk, outc_v.at[0, pl.ds(0, _TAIL // 4)])
        pltpu.sync_copy(
            outc_v.at[0, pl.ds(0, _TAIL // 4)],
            tpk.at[pl.ds(_NFULL * (_CHUNK // 4), _TAIL // 4)])


@jax.jit
def _repack(table_t, tail_pk):
    f = functools.partial(
        pl.kernel,
        out_type=jax.ShapeDtypeStruct((_PROWS, _PW), jnp.float32),
        mesh=_mesh(),
        scratch_types=[
            pltpu.VMEM((2, _DIM, _CHUNK), jnp.float32),
            pltpu.VMEM((_DIM, _CHUNK), jnp.float32),
            pltpu.VMEM((2, _CHUNK // 4, _PW), jnp.float32),
            pltpu.SemaphoreType.DMA,
            pltpu.SemaphoreType.DMA,
        ],
        compiler_params=_COMPILER_PARAMS,
    )(_repack_body)
    return f(table_t, tail_pk)


def _fused_body(tpk, embed_t, x_t, out_t, idx4_v, off_v, rows_v, slab, sem):
    wid = lax.axis_index("s") * _NC + lax.axis_index("c")
    col0 = wid * _BPW

    pltpu.sync_copy(x_t.at[:, pl.ds(col0, _BPW)], slab.at[pl.ds(0, _XDIM)])
    pltpu.sync_copy(embed_t.at[:, pl.ds(col0, _BPW)], idx4_v)

    def prep(t, _):
        l = t // (_BPW // 16)
        j = t % (_BPW // 16)
        v = idx4_v[l, pl.ds(j * 16, 16)]
        off_v[l, pl.ds(j * 16, 16)] = (v & 3) * _DIM
        idx4_v[l, pl.ds(j * 16, 16)] = v >> 2
        return ()

    lax.fori_loop(0, _L * (_BPW // 16), prep, (), unroll=False)

    def fire(l, par):
        pltpu.async_copy(tpk.at[idx4_v.at[l]], rows_v.at[par], sem)

    fire(0, 0)
    fire(1, 1)

    jvecs = [jj * 16 + lax.iota(jnp.int32, 16) for jj in range(_BPW // 16)]

    def step(l, par, row_off):
        pltpu.make_async_copy(
            tpk.at[idx4_v.at[l]], rows_v.at[par], sem).wait()
        rbase = _XDIM + l * _DIM - row_off
        for jj in range(_BPW // 16):
            off16 = off_v[l, pl.ds(jj * 16, 16)]

            @plsc.parallel_loop(0, _DIM, unroll=8)
            def _(rr):
                v = plsc.load_gather(rows_v.at[par], [jvecs[jj], off16 + rr])
                slab[rbase + rr, pl.ds(jj * 16, 16)] = v

        @pl.when(l + 2 < _L)
        def _():
            fire(l + 2, par)
        return ()

    def pair_a(k, _):
        step(2 * k, 0, 0)
        step(2 * k + 1, 1, 0)
        return ()

    lax.fori_loop(0, 4, pair_a, (), unroll=False)
    step(8, 0, 0)
    pltpu.sync_copy(slab, out_t.at[pl.ds(0, _HSLAB), pl.ds(col0, _BPW)])

    step(9, 1, _HSLAB)

    def pair_b(k, _):
        step(10 + 2 * k, 0, _HSLAB)
        step(11 + 2 * k, 1, _HSLAB)
        return ()

    lax.fori_loop(0, 5, pair_b, (), unroll=False)
    pltpu.sync_copy(slab, out_t.at[pl.ds(_HSLAB, _HSLAB), pl.ds(col0, _BPW)])


@jax.jit
def _fused(tpk, embed_t, x_t):
    f = functools.partial(
        pl.kernel,
        out_type=jax.ShapeDtypeStruct((_OUT, _B), jnp.float32),
        mesh=_mesh(),
        scratch_types=[
            pltpu.VMEM((_L, _BPW), jnp.int32),
            pltpu.VMEM((_L, _BPW), jnp.int32),
            pltpu.VMEM((2, _BPW, _PW), jnp.float32),
            pltpu.VMEM((_HSLAB, _BPW), jnp.float32),
            pltpu.SemaphoreType.DMA,
        ],
        compiler_params=_COMPILER_PARAMS,
    )(_fused_body)
    return f(tpk, embed_t, x_t)


def kernel(X, embed, table0):
    table_t = jnp.swapaxes(table0, 0, 1)
    tail_pk = jnp.reshape(
        lax.slice(table0, (_NFULL * _CHUNK, 0), (_VOCAB, _DIM)), (_TAIL // 4, _PW))
    tpk = _repack(table_t, tail_pk)
    embed_t = jnp.swapaxes(embed[0], 0, 1)
    x_t = jnp.swapaxes(X, 0, 1)
    out_t = _fused(tpk, embed_t, x_t)
    return jnp.swapaxes(out_t, 0, 1)
